# baseline (speedup 1.0000x reference)
import functools

import jax
import jax.numpy as jnp
from jax import lax
from jax.experimental import pallas as pl
from jax.experimental.pallas import tpu as pltpu
from jax.experimental.pallas import tpu_sc as plsc

BATCH = 4096
SEQ = 200
DIM = 128

_NC = 2
_NS = 16
_NW = _NC * _NS
_B_PER_W = BATCH // _NW

BB = 32


def _sc_gather_body(table_hbm, idx_hbm, out_hbm, idx_v, rows_v, sem):
    wid = lax.axis_index("s") * _NC + lax.axis_index("c")
    base = wid * _B_PER_W
    pltpu.sync_copy(idx_hbm.at[pl.ds(base, _B_PER_W)], idx_v)
    pltpu.async_copy(table_hbm.at[idx_v], rows_v, sem).wait()
    pltpu.sync_copy(rows_v, out_hbm.at[pl.ds(base, _B_PER_W)])


_sc_gather = functools.partial(
    pl.kernel,
    out_type=jax.ShapeDtypeStruct((BATCH, DIM), jnp.float32),
    mesh=plsc.VectorSubcoreMesh(core_axis_name="c", subcore_axis_name="s"),
    scratch_types=[
        pltpu.VMEM((_B_PER_W,), jnp.int32),
        pltpu.VMEM((_B_PER_W, DIM), jnp.float32),
        pltpu.SemaphoreType.DMA,
    ],
)(_sc_gather_body)


def _tc_add_body(te_ref, x_ref, o_ref):
    te = te_ref[...]
    o_ref[...] = x_ref[...] + te[:, None, :]


@jax.jit
def _tc_add(task_embed, inputs):
    return pl.pallas_call(
        _tc_add_body,
        grid=(BATCH // BB,),
        in_specs=[
            pl.BlockSpec((BB, DIM), lambda i: (i, 0)),
            pl.BlockSpec((BB, SEQ, DIM), lambda i: (i, 0, 0)),
        ],
        out_specs=pl.BlockSpec((BB, SEQ, DIM), lambda i: (i, 0, 0)),
        out_shape=jax.ShapeDtypeStruct((BATCH, SEQ, DIM), jnp.float32),
    )(task_embed, inputs)


def kernel(inputs, tasks, embeddings):
    tasks_i32 = tasks.astype(jnp.int32).reshape(-1)
    task_embed = _sc_gather(embeddings, tasks_i32)
    return _tc_add(task_embed, inputs)


# speedup vs baseline: 1.0173x; 1.0173x over previous
import functools

import jax
import jax.numpy as jnp
from jax import lax
from jax.experimental import pallas as pl
from jax.experimental.pallas import tpu as pltpu
from jax.experimental.pallas import tpu_sc as plsc

BATCH = 4096
SEQ = 200
DIM = 128

_NC = 2
_NS = 16
_NW = _NC * _NS
_B_PER_W = BATCH // _NW

BB = 64


def _sc_gather_body(table_hbm, idx_hbm, out_hbm, idx_v, rows_v, sem):
    wid = lax.axis_index("s") * _NC + lax.axis_index("c")
    base = wid * _B_PER_W
    pltpu.sync_copy(idx_hbm.at[pl.ds(base, _B_PER_W)], idx_v)
    pltpu.async_copy(table_hbm.at[idx_v], rows_v, sem).wait()
    pltpu.sync_copy(rows_v, out_hbm.at[pl.ds(base, _B_PER_W)])


_sc_gather = functools.partial(
    pl.kernel,
    out_type=jax.ShapeDtypeStruct((BATCH, DIM), jnp.float32),
    mesh=plsc.VectorSubcoreMesh(core_axis_name="c", subcore_axis_name="s"),
    scratch_types=[
        pltpu.VMEM((_B_PER_W,), jnp.int32),
        pltpu.VMEM((_B_PER_W, DIM), jnp.float32),
        pltpu.SemaphoreType.DMA,
    ],
)(_sc_gather_body)


def _tc_add_body(te_ref, x_ref, o_ref):
    te = te_ref[...]
    o_ref[...] = x_ref[...] + te[:, None, :]


@jax.jit
def _tc_add(task_embed, inputs):
    return pl.pallas_call(
        _tc_add_body,
        grid=(BATCH // BB,),
        in_specs=[
            pl.BlockSpec((BB, DIM), lambda i: (i, 0)),
            pl.BlockSpec((BB, SEQ, DIM), lambda i: (i, 0, 0)),
        ],
        out_specs=pl.BlockSpec((BB, SEQ, DIM), lambda i: (i, 0, 0)),
        out_shape=jax.ShapeDtypeStruct((BATCH, SEQ, DIM), jnp.float32),
    )(task_embed, inputs)


def kernel(inputs, tasks, embeddings):
    tasks_i32 = tasks.astype(jnp.int32).reshape(-1)
    task_embed = _sc_gather(embeddings, tasks_i32)
    return _tc_add(task_embed, inputs)


# speedup vs baseline: 1.0244x; 1.0070x over previous
import functools

import jax
import jax.numpy as jnp
from jax import lax
from jax.experimental import pallas as pl
from jax.experimental.pallas import tpu as pltpu
from jax.experimental.pallas import tpu_sc as plsc

BATCH = 4096
SEQ = 200
DIM = 128

_NC = 2
_NS = 16
_NW = _NC * _NS
_B_PER_W = BATCH // _NW

BB = 128


def _sc_gather_body(table_hbm, idx_hbm, out_hbm, idx_v, rows_v, sem):
    wid = lax.axis_index("s") * _NC + lax.axis_index("c")
    base = wid * _B_PER_W
    pltpu.sync_copy(idx_hbm.at[pl.ds(base, _B_PER_W)], idx_v)
    pltpu.async_copy(table_hbm.at[idx_v], rows_v, sem).wait()
    pltpu.sync_copy(rows_v, out_hbm.at[pl.ds(base, _B_PER_W)])


_sc_gather = functools.partial(
    pl.kernel,
    out_type=jax.ShapeDtypeStruct((BATCH, DIM), jnp.float32),
    mesh=plsc.VectorSubcoreMesh(core_axis_name="c", subcore_axis_name="s"),
    scratch_types=[
        pltpu.VMEM((_B_PER_W,), jnp.int32),
        pltpu.VMEM((_B_PER_W, DIM), jnp.float32),
        pltpu.SemaphoreType.DMA,
    ],
)(_sc_gather_body)


def _tc_add_body(te_ref, x_ref, o_ref):
    te = te_ref[...]
    o_ref[...] = x_ref[...] + te[:, None, :]


@jax.jit
def _tc_add(task_embed, inputs):
    return pl.pallas_call(
        _tc_add_body,
        grid=(BATCH // BB,),
        in_specs=[
            pl.BlockSpec((BB, DIM), lambda i: (i, 0)),
            pl.BlockSpec((BB, SEQ, DIM), lambda i: (i, 0, 0)),
        ],
        out_specs=pl.BlockSpec((BB, SEQ, DIM), lambda i: (i, 0, 0)),
        out_shape=jax.ShapeDtypeStruct((BATCH, SEQ, DIM), jnp.float32),
    )(task_embed, inputs)


def kernel(inputs, tasks, embeddings):
    tasks_i32 = tasks.astype(jnp.int32).reshape(-1)
    task_embed = _sc_gather(embeddings, tasks_i32)
    return _tc_add(task_embed, inputs)


# speedup vs baseline: 1.0877x; 1.0617x over previous
import functools

import jax
import jax.numpy as jnp
from jax import lax
from jax.experimental import pallas as pl
from jax.experimental.pallas import tpu as pltpu
from jax.experimental.pallas import tpu_sc as plsc

BATCH = 4096
SEQ = 200
DIM = 128

_NC = 2
_NS = 16
_NW = _NC * _NS
_B_PER_W = BATCH // _NW

BB = 128


def _sc_gather_body(table_hbm, idx_hbm, out_hbm, idx_v, rows_v, sem):
    wid = lax.axis_index("s") * _NC + lax.axis_index("c")
    base = wid * _B_PER_W
    pltpu.sync_copy(idx_hbm.at[pl.ds(base, _B_PER_W)], idx_v)
    pltpu.async_copy(table_hbm.at[idx_v], rows_v, sem).wait()
    pltpu.sync_copy(rows_v, out_hbm.at[pl.ds(base, _B_PER_W)])


_sc_gather = functools.partial(
    pl.kernel,
    out_type=jax.ShapeDtypeStruct((BATCH, DIM), jnp.float32),
    mesh=plsc.VectorSubcoreMesh(core_axis_name="c", subcore_axis_name="s"),
    scratch_types=[
        pltpu.VMEM((_B_PER_W,), jnp.int32),
        pltpu.VMEM((_B_PER_W, DIM), jnp.float32),
        pltpu.SemaphoreType.DMA,
    ],
)(_sc_gather_body)


def _tc_add_body(te_ref, x_ref, o_ref):
    te = te_ref[...]
    o_ref[...] = x_ref[...] + te[:, None, :]


@jax.jit
def _tc_add(task_embed, inputs):
    return pl.pallas_call(
        _tc_add_body,
        grid=(BATCH // BB,),
        in_specs=[
            pl.BlockSpec((BB, DIM), lambda i: (i, 0)),
            pl.BlockSpec((BB, SEQ, DIM), lambda i: (i, 0, 0)),
        ],
        out_specs=pl.BlockSpec((BB, SEQ, DIM), lambda i: (i, 0, 0)),
        out_shape=jax.ShapeDtypeStruct((BATCH, SEQ, DIM), jnp.float32),
    )(task_embed, inputs)


def _tc_fused_body(t_ref, emb_ref, x_ref, o_ref):
    t = t_ref[...]
    onehot = (t == lax.broadcasted_iota(jnp.int32, (BB, 1000), 1)).astype(
        jnp.float32)
    te = jnp.dot(onehot, emb_ref[...], preferred_element_type=jnp.float32)
    o_ref[...] = x_ref[...] + te[:, None, :]


@jax.jit
def _tc_fused(tasks_i32, embeddings, inputs):
    return pl.pallas_call(
        _tc_fused_body,
        grid=(BATCH // BB,),
        in_specs=[
            pl.BlockSpec((BB, 1), lambda i: (i, 0)),
            pl.BlockSpec((1000, DIM), lambda i: (0, 0)),
            pl.BlockSpec((BB, SEQ, DIM), lambda i: (i, 0, 0)),
        ],
        out_specs=pl.BlockSpec((BB, SEQ, DIM), lambda i: (i, 0, 0)),
        out_shape=jax.ShapeDtypeStruct((BATCH, SEQ, DIM), jnp.float32),
    )(tasks_i32, embeddings, inputs)


def kernel(inputs, tasks, embeddings):
    tasks_i32 = tasks.astype(jnp.int32)
    return _tc_fused(tasks_i32, embeddings, inputs)
